# initial kernel scaffold (unmeasured)
import jax
import jax.numpy as jnp
from jax import lax
from jax.experimental import pallas as pl
from jax.experimental.pallas import tpu as pltpu

N_DEV = 4
M_PER = 2048
HALF = M_PER // 2
K = 8192
N_PER = 1024


def _gather_body(x_ref, gat_ref, sa_send, sa_recv, sb_send, sb_recv, cp_sem):
    my = lax.axis_index("i")
    right = lax.rem(my + 1, N_DEV)
    left = lax.rem(my + N_DEV - 1, N_DEV)

    barrier = pltpu.get_barrier_semaphore()
    for nbr in (left, right):
        pl.semaphore_signal(
            barrier, inc=1, device_id=(nbr,),
            device_id_type=pl.DeviceIdType.MESH,
        )
    pl.semaphore_wait(barrier, 2)

    cp = pltpu.make_async_copy(
        x_ref, gat_ref.at[pl.ds(my * M_PER, M_PER), :], cp_sem
    )
    cp.start()

    for h in range(N_DEV - 1):
        oa = lax.rem(my - h + N_DEV, N_DEV)
        ob = lax.rem(my + h, N_DEV)
        if h == 0:
            src_a = x_ref.at[pl.ds(0, HALF), :]
            src_b = x_ref.at[pl.ds(HALF, HALF), :]
        else:
            src_a = gat_ref.at[pl.ds(oa * M_PER, HALF), :]
            src_b = gat_ref.at[pl.ds(ob * M_PER + HALF, HALF), :]
        rdma_a = pltpu.make_async_remote_copy(
            src_ref=src_a,
            dst_ref=gat_ref.at[pl.ds(oa * M_PER, HALF), :],
            send_sem=sa_send.at[h],
            recv_sem=sa_recv.at[h],
            device_id=(right,),
            device_id_type=pl.DeviceIdType.MESH,
        )
        rdma_b = pltpu.make_async_remote_copy(
            src_ref=src_b,
            dst_ref=gat_ref.at[pl.ds(ob * M_PER + HALF, HALF), :],
            send_sem=sb_send.at[h],
            recv_sem=sb_recv.at[h],
            device_id=(left,),
            device_id_type=pl.DeviceIdType.MESH,
        )
        rdma_a.start()
        rdma_b.start()
        rdma_a.wait()
        rdma_b.wait()

    cp.wait()


def _all_gather(x_bf16):
    return pl.pallas_call(
        _gather_body,
        out_shape=jax.ShapeDtypeStruct((N_DEV * M_PER, K), jnp.bfloat16),
        in_specs=[pl.BlockSpec(memory_space=pltpu.ANY)],
        out_specs=pl.BlockSpec(memory_space=pltpu.ANY),
        scratch_shapes=[
            pltpu.SemaphoreType.DMA((N_DEV - 1,)),
            pltpu.SemaphoreType.DMA((N_DEV - 1,)),
            pltpu.SemaphoreType.DMA((N_DEV - 1,)),
            pltpu.SemaphoreType.DMA((N_DEV - 1,)),
            pltpu.SemaphoreType.DMA,
        ],
        compiler_params=pltpu.CompilerParams(collective_id=0),
    )(x_bf16)


def _gemm_body(xg_ref, w_ref, o_ref):
    y = jnp.dot(xg_ref[...], w_ref[...], preferred_element_type=jnp.float32)
    o_ref[...] = y * jax.nn.sigmoid(y)


def _gemm_silu(xg, w_bf16):
    m_blk = 512
    return pl.pallas_call(
        _gemm_body,
        grid=(N_DEV * M_PER // m_blk,),
        in_specs=[
            pl.BlockSpec((m_blk, K), lambda i: (i, 0)),
            pl.BlockSpec((K, N_PER), lambda i: (0, 0)),
        ],
        out_specs=pl.BlockSpec((m_blk, N_PER), lambda i: (i, 0)),
        out_shape=jax.ShapeDtypeStruct((N_DEV * M_PER, N_PER), jnp.float32),
        compiler_params=pltpu.CompilerParams(
            dimension_semantics=("arbitrary",)
        ),
    )(xg, w_bf16)


def kernel(x, w_mat):
    x_bf16 = x.astype(jnp.bfloat16)
    w_bf16 = w_mat.astype(jnp.bfloat16)
    xg = _all_gather(x_bf16)
    return _gemm_silu(xg, w_bf16)


# baseline (device time: 1222156 ns/iter reference)
import jax
import jax.numpy as jnp
from jax import lax
from jax.experimental import pallas as pl
from jax.experimental.pallas import tpu as pltpu

N_DEV = 4
M_PER = 2048
HALF = M_PER // 2
K = 8192
N_PER = 1024


def _gather_body(x_ref, gat_ref, sa_send, sa_recv, sb_send, sb_recv, cp_sem):
    my = lax.axis_index("i")
    right = lax.rem(my + 1, N_DEV)
    left = lax.rem(my + N_DEV - 1, N_DEV)

    barrier = pltpu.get_barrier_semaphore()
    for nbr in (left, right):
        pl.semaphore_signal(
            barrier, inc=1, device_id=(nbr,),
            device_id_type=pl.DeviceIdType.MESH,
        )
    pl.semaphore_wait(barrier, 2)

    cp = pltpu.make_async_copy(
        x_ref, gat_ref.at[pl.ds(my * M_PER, M_PER), :], cp_sem
    )
    cp.start()

    for h in range(N_DEV - 1):
        oa = lax.rem(my - h + N_DEV, N_DEV)
        ob = lax.rem(my + h, N_DEV)
        if h == 0:
            src_a = x_ref.at[pl.ds(0, HALF), :]
            src_b = x_ref.at[pl.ds(HALF, HALF), :]
        else:
            src_a = gat_ref.at[pl.ds(oa * M_PER, HALF), :]
            src_b = gat_ref.at[pl.ds(ob * M_PER + HALF, HALF), :]
        rdma_a = pltpu.make_async_remote_copy(
            src_ref=src_a,
            dst_ref=gat_ref.at[pl.ds(oa * M_PER, HALF), :],
            send_sem=sa_send.at[h],
            recv_sem=sa_recv.at[h],
            device_id=(right,),
            device_id_type=pl.DeviceIdType.MESH,
        )
        rdma_b = pltpu.make_async_remote_copy(
            src_ref=src_b,
            dst_ref=gat_ref.at[pl.ds(ob * M_PER + HALF, HALF), :],
            send_sem=sb_send.at[h],
            recv_sem=sb_recv.at[h],
            device_id=(left,),
            device_id_type=pl.DeviceIdType.MESH,
        )
        rdma_a.start()
        rdma_b.start()
        rdma_a.wait()
        rdma_b.wait()

    cp.wait()


def _all_gather(x_bf16):
    return pl.pallas_call(
        _gather_body,
        out_shape=jax.ShapeDtypeStruct((N_DEV * M_PER, K), jnp.bfloat16),
        in_specs=[pl.BlockSpec(memory_space=pl.ANY)],
        out_specs=pl.BlockSpec(memory_space=pl.ANY),
        scratch_shapes=[
            pltpu.SemaphoreType.DMA((N_DEV - 1,)),
            pltpu.SemaphoreType.DMA((N_DEV - 1,)),
            pltpu.SemaphoreType.DMA((N_DEV - 1,)),
            pltpu.SemaphoreType.DMA((N_DEV - 1,)),
            pltpu.SemaphoreType.DMA,
        ],
        compiler_params=pltpu.CompilerParams(collective_id=0),
    )(x_bf16)


def _gemm_body(xg_ref, w_ref, o_ref):
    y = jnp.dot(xg_ref[...], w_ref[...], preferred_element_type=jnp.float32)
    o_ref[...] = y * jax.nn.sigmoid(y)


def _gemm_silu(xg, w_bf16):
    m_blk = 512
    return pl.pallas_call(
        _gemm_body,
        grid=(N_DEV * M_PER // m_blk,),
        in_specs=[
            pl.BlockSpec((m_blk, K), lambda i: (i, 0)),
            pl.BlockSpec((K, N_PER), lambda i: (0, 0)),
        ],
        out_specs=pl.BlockSpec((m_blk, N_PER), lambda i: (i, 0)),
        out_shape=jax.ShapeDtypeStruct((N_DEV * M_PER, N_PER), jnp.float32),
        compiler_params=pltpu.CompilerParams(
            dimension_semantics=("arbitrary",)
        ),
    )(xg, w_bf16)


def kernel(x, w_mat):
    x_bf16 = x.astype(jnp.bfloat16)
    w_bf16 = w_mat.astype(jnp.bfloat16)
    xg = _all_gather(x_bf16)
    return _gemm_silu(xg, w_bf16)
